# baseline (speedup 1.0000x reference)
import functools

import jax
import jax.numpy as jnp
from jax import lax
from jax.experimental import pallas as pl
from jax.experimental.pallas import tpu as pltpu
from jax.experimental.pallas import tpu_sc as plsc

NC = 2
NS = 16
NT = NC * NS

_INTERPRET = False


def _sc_mesh():
    return plsc.VectorSubcoreMesh(core_axis_name="c", subcore_axis_name="s",
                                  num_cores=NC, num_subcores=NS)


def _drain_partition(n_nodes):
    rpt = -(-(n_nodes // NS) // 8) * 8
    last = n_nodes - (NS - 1) * rpt
    assert last > 0 and last % 8 == 0 and n_nodes % 8 == 0
    return rpt, last


def _make_deg_kernel(n_edges, n_nodes, chunk):
    n_chunks = n_edges // chunk
    nj = -(-n_chunks // NS)
    rpt, last = _drain_partition(n_nodes)

    @functools.partial(
        pl.kernel,
        out_type=[jax.ShapeDtypeStruct((NC * n_nodes,), jnp.float32)],
        mesh=_sc_mesh(),
        interpret=_INTERPRET,
        scratch_types=[
            [pltpu.VMEM((chunk,), jnp.int32)] * 4,
            pltpu.VMEM((chunk,), jnp.float32),
            pltpu.VMEM((rpt,), jnp.float32),
            pltpu.MemorySpace.VMEM_SHARED((n_nodes,), jnp.float32),
            [pltpu.SemaphoreType.DMA] * 4,
            [pltpu.SemaphoreType.DMA] * 4,
        ],
    )
    def deg_kernel(src_hbm, dst_hbm, zeros_hbm,
                   deg_hbm,
                   idx_v, ones_v, bounce, acc, isem, ssem):
        c = lax.axis_index("c")
        s = lax.axis_index("s")
        r0 = s * rpt

        for k in range(chunk // 16):
            ones_v[pl.ds(k * 16, 16)] = jnp.ones((16,), jnp.float32)

        @pl.when(s < NS - 1)
        def _():
            pltpu.sync_copy(zeros_hbm.at[pl.ds(r0, rpt)], bounce)
            pltpu.sync_copy(bounce, acc.at[pl.ds(r0, rpt)])

        @pl.when(s == NS - 1)
        def _():
            pltpu.sync_copy(zeros_hbm.at[pl.ds(r0, last)],
                            bounce.at[pl.ds(0, last)])
            pltpu.sync_copy(bounce.at[pl.ds(0, last)],
                            acc.at[pl.ds(r0, last)])

        plsc.subcore_barrier()

        def run(list_hbm):
            def issue_idx(jj, b):
                pltpu.async_copy(
                    list_hbm.at[pl.ds((s + NS * jj) * chunk, chunk)],
                    idx_v[b], isem[b])

            def wait_scatter(b):
                pltpu.make_async_copy(ones_v, acc.at[idx_v[b]],
                                      ssem[b]).wait()

            def scatter(b):
                pltpu.make_async_copy(
                    list_hbm.at[pl.ds(0, chunk)], idx_v[b], isem[b]).wait()
                pltpu.async_copy(ones_v, acc.at[idx_v[b]], ssem[b],
                                 add=True)

            issue_idx(0, 0)

            @pl.when(s + NS < n_chunks)
            def _():
                issue_idx(1, 1)

            def body(i, carry):
                for b in (0, 1, 2, 3):
                    jj = 4 * i + b
                    gm2 = s + NS * (jj - 2)

                    @pl.when((jj >= 2) & (gm2 < n_chunks))
                    def _():
                        wait_scatter((b + 2) % 4)

                    g2 = s + NS * (jj + 2)

                    @pl.when(g2 < n_chunks)
                    def _():
                        issue_idx(jj + 2, (b + 2) % 4)

                    g = s + NS * jj

                    @pl.when(g < n_chunks)
                    def _():
                        scatter(b)
                return carry

            n_outer = (nj + 4) // 4
            lax.fori_loop(0, n_outer, body, 0)
            jmax = 4 * n_outer
            for jj in (jmax - 2, jmax - 1):
                @pl.when(s + NS * jj < n_chunks)
                def _():
                    wait_scatter(jj % 4)

        @pl.when(c == 0)
        def _():
            run(src_hbm)

        @pl.when(c == 1)
        def _():
            run(dst_hbm)

        plsc.subcore_barrier()

        @pl.when(s < NS - 1)
        def _():
            pltpu.sync_copy(acc.at[pl.ds(r0, rpt)], bounce)
            pltpu.sync_copy(bounce,
                            deg_hbm.at[pl.ds(c * n_nodes + r0, rpt)])

        @pl.when(s == NS - 1)
        def _():
            pltpu.sync_copy(acc.at[pl.ds(r0, last)],
                            bounce.at[pl.ds(0, last)])
            pltpu.sync_copy(bounce.at[pl.ds(0, last)],
                            deg_hbm.at[pl.ds(c * n_nodes + r0, last)])

    return deg_kernel


def _make_spmm_kernel(n_edges, n_nodes, d, chunk):
    n_chunks = n_edges // chunk
    nj = -(-n_chunks // NT)
    n_outer = (nj + 4) // 4
    rpt, last = _drain_partition(n_nodes)

    @functools.partial(
        pl.kernel,
        out_type=[jax.ShapeDtypeStruct((NC, n_nodes, d), jnp.float32)],
        mesh=_sc_mesh(),
        interpret=_INTERPRET,
        scratch_types=[
            [pltpu.VMEM((chunk,), jnp.int32)] * 4,
            [pltpu.VMEM((chunk,), jnp.int32)] * 4,
            [pltpu.VMEM((chunk, d), jnp.float32)] * 4,
            pltpu.MemorySpace.VMEM_SHARED((n_nodes, d), jnp.float32),
            [pltpu.SemaphoreType.DMA] * 4,
            [pltpu.SemaphoreType.DMA] * 4,
            [pltpu.SemaphoreType.DMA] * 4,
        ],
    )
    def spmm_kernel(table_hbm, src_hbm, dst_hbm, zeros_hbm,
                    agg_hbm,
                    idx_s, idx_d, rows_v, acc, gsem, isem, ssem):
        c = lax.axis_index("c")
        s = lax.axis_index("s")
        t = c * NS + s
        r0 = s * rpt

        def issue_idx(g, b):
            off = g * chunk
            pltpu.async_copy(src_hbm.at[pl.ds(off, chunk)], idx_s[b],
                             isem[b])
            pltpu.async_copy(dst_hbm.at[pl.ds(off, chunk)], idx_d[b],
                             isem[b])

        def wait_idx_issue_gather(b):
            pltpu.make_async_copy(src_hbm.at[pl.ds(0, chunk)], idx_s[b],
                                  isem[b]).wait()
            pltpu.make_async_copy(dst_hbm.at[pl.ds(0, chunk)], idx_d[b],
                                  isem[b]).wait()
            pltpu.async_copy(table_hbm.at[idx_s[b]], rows_v[b], gsem[b])

        def wait_scatter(b):
            pltpu.make_async_copy(rows_v[b], acc.at[idx_d[b]],
                                  ssem[b]).wait()

        def finish(b):
            pltpu.make_async_copy(table_hbm.at[idx_s[b]], rows_v[b],
                                  gsem[b]).wait()
            pltpu.async_copy(rows_v[b], acc.at[idx_d[b]], ssem[b],
                             add=True)

        issue_idx(t, 0)

        @pl.when(t + NT < n_chunks)
        def _():
            issue_idx(t + NT, 1)

        wait_idx_issue_gather(0)

        @pl.when(s < NS - 1)
        def _():
            pltpu.sync_copy(zeros_hbm.at[pl.ds(r0, rpt)],
                            acc.at[pl.ds(r0, rpt)])

        @pl.when(s == NS - 1)
        def _():
            pltpu.sync_copy(zeros_hbm.at[pl.ds(r0, last)],
                            acc.at[pl.ds(r0, last)])

        plsc.subcore_barrier()

        def outer(i, carry):
            for b in (0, 1, 2, 3):
                j = 4 * i + b
                gm2 = t + NT * (j - 2)

                @pl.when((j >= 2) & (gm2 < n_chunks))
                def _():
                    wait_scatter((b + 2) % 4)

                g2 = t + NT * (j + 2)

                @pl.when(g2 < n_chunks)
                def _():
                    issue_idx(g2, (b + 2) % 4)

                g1 = t + NT * (j + 1)

                @pl.when(g1 < n_chunks)
                def _():
                    wait_idx_issue_gather((b + 1) % 4)

                g = t + NT * j

                @pl.when(g < n_chunks)
                def _():
                    finish(b)
            return carry

        lax.fori_loop(0, n_outer, outer, 0)
        jmax = 4 * n_outer
        for j in (jmax - 2, jmax - 1):
            @pl.when(t + NT * j < n_chunks)
            def _():
                wait_scatter(j % 4)
        plsc.subcore_barrier()

        r0 = s * rpt

        @pl.when(s < NS - 1)
        def _():
            pltpu.sync_copy(acc.at[pl.ds(r0, rpt)],
                            agg_hbm.at[c, pl.ds(r0, rpt)])

        @pl.when(s == NS - 1)
        def _():
            pltpu.sync_copy(acc.at[pl.ds(r0, last)],
                            agg_hbm.at[c, pl.ds(r0, last)])

    return spmm_kernel


def _norm_from_deg(deg_ref):
    deg = deg_ref[...]
    return jnp.where(deg > 0.0,
                     jnp.power(jnp.maximum(deg, 1.0), -0.5), 0.0)


def _pre_body(x_ref, dego_ref, o_ref):
    o_ref[...] = x_ref[...] * _norm_from_deg(dego_ref)


def _mid_body(agg_ref, degi_ref, dego_ref, w_ref, b_ref, o_ref):
    norm_in = _norm_from_deg(degi_ref)
    norm_out = _norm_from_deg(dego_ref)
    rst = (agg_ref[0] + agg_ref[1]) * norm_in
    h = jax.nn.relu(jnp.dot(rst, w_ref[...],
                            preferred_element_type=jnp.float32,
                            precision=lax.Precision.HIGHEST) + b_ref[...])
    o_ref[...] = h * norm_out


def _post_body(agg_ref, degi_ref, w_ref, b_ref, wd1_ref, bd1_ref, wd2_ref,
               bd2_ref, starts_ref, counts_ref, o_ref):
    norm_in = _norm_from_deg(degi_ref)
    rst = (agg_ref[0] + agg_ref[1]) * norm_in
    h = jax.nn.relu(jnp.dot(rst, w_ref[...],
                            preferred_element_type=jnp.float32,
                            precision=lax.Precision.HIGHEST) + b_ref[...])
    n = h.shape[0]
    ng = starts_ref.shape[0]
    col = lax.broadcasted_iota(jnp.int32, (ng, n), 1).astype(jnp.float32)
    starts = starts_ref[...]
    counts = counts_ref[...]
    sel = jnp.where((col >= starts) & (col < starts + counts), 1.0, 0.0)
    sums = jnp.dot(sel, h, preferred_element_type=jnp.float32,
                 precision=lax.Precision.HIGHEST)
    means = sums / jnp.maximum(counts, 1.0)
    z = jax.nn.relu(
        jnp.dot(means, wd1_ref[...], preferred_element_type=jnp.float32,
                 precision=lax.Precision.HIGHEST)
        + bd1_ref[...])
    o_ref[...] = (jnp.dot(z, wd2_ref[...], preferred_element_type=jnp.float32,
                 precision=lax.Precision.HIGHEST)
                  + bd2_ref[...])


def kernel(node_features, edge_features, pair_indices, num_nodes, num_edges,
           W_gcn0, b_gcn0, W_gcn1, b_gcn1, W_d1, b_d1, W_d2, b_d2):
    del edge_features, num_edges
    n_nodes, _ = node_features.shape
    n_edges = pair_indices.shape[1]
    hidden = W_gcn0.shape[1]
    ng = num_nodes.shape[0]

    deg_chunk = 128
    spmm_chunk = 80
    assert n_edges % deg_chunk == 0 and n_edges % spmm_chunk == 0

    src = pair_indices[0]
    dst = pair_indices[1]
    zeros_tab = jnp.zeros((n_nodes, hidden), jnp.float32)
    zeros_deg = jnp.zeros((n_nodes,), jnp.float32)

    deg_call = _make_deg_kernel(n_edges, n_nodes, deg_chunk)
    spmm_call = _make_spmm_kernel(n_edges, n_nodes, hidden, spmm_chunk)

    (deg1d,) = deg_call(src, dst, zeros_deg)
    dego_col = deg1d[:n_nodes].reshape(n_nodes, 1)
    degi_col = deg1d[n_nodes:].reshape(n_nodes, 1)

    m0 = pl.pallas_call(
        _pre_body,
        interpret=_INTERPRET,
        out_shape=jax.ShapeDtypeStruct((n_nodes, hidden), jnp.float32),
    )(node_features, dego_col)
    (agg0,) = spmm_call(m0, src, dst, zeros_tab)

    m1 = pl.pallas_call(
        _mid_body,
        interpret=_INTERPRET,
        out_shape=jax.ShapeDtypeStruct((n_nodes, hidden), jnp.float32),
    )(agg0, degi_col, dego_col, W_gcn0, b_gcn0.reshape(1, hidden))
    (agg1,) = spmm_call(m1, src, dst, zeros_tab)

    counts = num_nodes.astype(jnp.float32).reshape(ng, 1)
    starts = jnp.concatenate(
        [jnp.zeros((1,), jnp.float32),
         jnp.cumsum(num_nodes.astype(jnp.float32))[:-1]]).reshape(ng, 1)
    pred_hidden = W_d1.shape[1]
    out = pl.pallas_call(
        _post_body,
        interpret=_INTERPRET,
        out_shape=jax.ShapeDtypeStruct((ng, 1), jnp.float32),
    )(agg1, degi_col, W_gcn1, b_gcn1.reshape(1, hidden), W_d1,
      b_d1.reshape(1, pred_hidden), W_d2, b_d2.reshape(1, 1),
      starts, counts)
    return out


# speedup vs baseline: 1.0240x; 1.0240x over previous
import functools

import jax
import jax.numpy as jnp
from jax import lax
from jax.experimental import pallas as pl
from jax.experimental.pallas import tpu as pltpu
from jax.experimental.pallas import tpu_sc as plsc

NC = 2
NS = 16
NT = NC * NS

def _sc_mesh():
    return plsc.VectorSubcoreMesh(core_axis_name="c", subcore_axis_name="s",
                                  num_cores=NC, num_subcores=NS)


def _drain_partition(n_nodes):
    rpt = -(-(n_nodes // NS) // 8) * 8
    last = n_nodes - (NS - 1) * rpt
    assert last > 0 and last % 8 == 0 and n_nodes % 8 == 0
    return rpt, last


def _make_deg_kernel(n_edges, n_nodes, chunk):
    n_chunks = n_edges // chunk
    nj = -(-n_chunks // NS)
    rpt, last = _drain_partition(n_nodes)

    @functools.partial(
        pl.kernel,
        out_type=[jax.ShapeDtypeStruct((NC * n_nodes,), jnp.float32)],
        mesh=_sc_mesh(),
        scratch_types=[
            [pltpu.VMEM((chunk,), jnp.int32)] * 4,
            pltpu.VMEM((chunk,), jnp.float32),
            pltpu.VMEM((rpt,), jnp.float32),
            pltpu.MemorySpace.VMEM_SHARED((n_nodes,), jnp.float32),
            [pltpu.SemaphoreType.DMA] * 4,
            [pltpu.SemaphoreType.DMA] * 4,
        ],
    )
    def deg_kernel(src_hbm, dst_hbm, zeros_hbm,
                   deg_hbm,
                   idx_v, ones_v, bounce, acc, isem, ssem):
        c = lax.axis_index("c")
        s = lax.axis_index("s")
        r0 = s * rpt

        for k in range(chunk // 16):
            ones_v[pl.ds(k * 16, 16)] = jnp.ones((16,), jnp.float32)

        @pl.when(s < NS - 1)
        def _():
            pltpu.sync_copy(zeros_hbm.at[pl.ds(r0, rpt)], bounce)
            pltpu.sync_copy(bounce, acc.at[pl.ds(r0, rpt)])

        @pl.when(s == NS - 1)
        def _():
            pltpu.sync_copy(zeros_hbm.at[pl.ds(r0, last)],
                            bounce.at[pl.ds(0, last)])
            pltpu.sync_copy(bounce.at[pl.ds(0, last)],
                            acc.at[pl.ds(r0, last)])

        plsc.subcore_barrier()

        def run(list_hbm):
            def issue_idx(jj, b):
                pltpu.async_copy(
                    list_hbm.at[pl.ds((s + NS * jj) * chunk, chunk)],
                    idx_v[b], isem[b])

            def wait_scatter(b):
                pltpu.make_async_copy(ones_v, acc.at[idx_v[b]],
                                      ssem[b]).wait()

            def scatter(b):
                pltpu.make_async_copy(
                    list_hbm.at[pl.ds(0, chunk)], idx_v[b], isem[b]).wait()
                pltpu.async_copy(ones_v, acc.at[idx_v[b]], ssem[b],
                                 add=True)

            issue_idx(0, 0)

            @pl.when(s + NS < n_chunks)
            def _():
                issue_idx(1, 1)

            def body(i, carry):
                for b in (0, 1, 2, 3):
                    jj = 4 * i + b
                    gm2 = s + NS * (jj - 2)

                    @pl.when((jj >= 2) & (gm2 < n_chunks))
                    def _():
                        wait_scatter((b + 2) % 4)

                    g2 = s + NS * (jj + 2)

                    @pl.when(g2 < n_chunks)
                    def _():
                        issue_idx(jj + 2, (b + 2) % 4)

                    g = s + NS * jj

                    @pl.when(g < n_chunks)
                    def _():
                        scatter(b)
                return carry

            n_outer = (nj + 4) // 4
            lax.fori_loop(0, n_outer, body, 0)
            jmax = 4 * n_outer
            for jj in (jmax - 2, jmax - 1):
                @pl.when(s + NS * jj < n_chunks)
                def _():
                    wait_scatter(jj % 4)

        @pl.when(c == 0)
        def _():
            run(src_hbm)

        @pl.when(c == 1)
        def _():
            run(dst_hbm)

        plsc.subcore_barrier()

        @pl.when(s < NS - 1)
        def _():
            pltpu.sync_copy(acc.at[pl.ds(r0, rpt)], bounce)
            pltpu.sync_copy(bounce,
                            deg_hbm.at[pl.ds(c * n_nodes + r0, rpt)])

        @pl.when(s == NS - 1)
        def _():
            pltpu.sync_copy(acc.at[pl.ds(r0, last)],
                            bounce.at[pl.ds(0, last)])
            pltpu.sync_copy(bounce.at[pl.ds(0, last)],
                            deg_hbm.at[pl.ds(c * n_nodes + r0, last)])

    return deg_kernel


def _make_spmm_kernel(n_edges, n_nodes, d, chunk):
    n_chunks = n_edges // chunk
    nj = -(-n_chunks // NT)
    n_outer = (nj + 4) // 4
    rpt, last = _drain_partition(n_nodes)

    @functools.partial(
        pl.kernel,
        out_type=[jax.ShapeDtypeStruct((NC, n_nodes, d), jnp.float32)],
        mesh=_sc_mesh(),
        scratch_types=[
            [pltpu.VMEM((chunk,), jnp.int32)] * 4,
            [pltpu.VMEM((chunk,), jnp.int32)] * 4,
            [pltpu.VMEM((chunk, d), jnp.float32)] * 4,
            pltpu.MemorySpace.VMEM_SHARED((n_nodes, d), jnp.float32),
            [pltpu.SemaphoreType.DMA] * 4,
            [pltpu.SemaphoreType.DMA] * 4,
            [pltpu.SemaphoreType.DMA] * 4,
        ],
    )
    def spmm_kernel(table_hbm, src_hbm, dst_hbm, zeros_hbm,
                    agg_hbm,
                    idx_s, idx_d, rows_v, acc, gsem, isem, ssem):
        c = lax.axis_index("c")
        s = lax.axis_index("s")
        t = c * NS + s
        r0 = s * rpt

        def issue_idx(g, b):
            off = g * chunk
            pltpu.async_copy(src_hbm.at[pl.ds(off, chunk)], idx_s[b],
                             isem[b])
            pltpu.async_copy(dst_hbm.at[pl.ds(off, chunk)], idx_d[b],
                             isem[b])

        def wait_idx_issue_gather(b):
            pltpu.make_async_copy(src_hbm.at[pl.ds(0, chunk)], idx_s[b],
                                  isem[b]).wait()
            pltpu.make_async_copy(dst_hbm.at[pl.ds(0, chunk)], idx_d[b],
                                  isem[b]).wait()
            pltpu.async_copy(table_hbm.at[idx_s[b]], rows_v[b], gsem[b])

        def wait_scatter(b):
            pltpu.make_async_copy(rows_v[b], acc.at[idx_d[b]],
                                  ssem[b]).wait()

        def finish(b):
            pltpu.make_async_copy(table_hbm.at[idx_s[b]], rows_v[b],
                                  gsem[b]).wait()
            pltpu.async_copy(rows_v[b], acc.at[idx_d[b]], ssem[b],
                             add=True)

        issue_idx(t, 0)

        @pl.when(t + NT < n_chunks)
        def _():
            issue_idx(t + NT, 1)

        wait_idx_issue_gather(0)

        @pl.when(s < NS - 1)
        def _():
            pltpu.sync_copy(zeros_hbm.at[pl.ds(r0, rpt)],
                            acc.at[pl.ds(r0, rpt)])

        @pl.when(s == NS - 1)
        def _():
            pltpu.sync_copy(zeros_hbm.at[pl.ds(r0, last)],
                            acc.at[pl.ds(r0, last)])

        plsc.subcore_barrier()

        def outer(i, carry):
            for b in (0, 1, 2, 3):
                j = 4 * i + b
                gm2 = t + NT * (j - 2)

                @pl.when((j >= 2) & (gm2 < n_chunks))
                def _():
                    wait_scatter((b + 2) % 4)

                g2 = t + NT * (j + 2)

                @pl.when(g2 < n_chunks)
                def _():
                    issue_idx(g2, (b + 2) % 4)

                g1 = t + NT * (j + 1)

                @pl.when(g1 < n_chunks)
                def _():
                    wait_idx_issue_gather((b + 1) % 4)

                g = t + NT * j

                @pl.when(g < n_chunks)
                def _():
                    finish(b)
            return carry

        lax.fori_loop(0, n_outer, outer, 0)
        jmax = 4 * n_outer
        for j in (jmax - 2, jmax - 1):
            @pl.when(t + NT * j < n_chunks)
            def _():
                wait_scatter(j % 4)
        plsc.subcore_barrier()

        r0 = s * rpt

        @pl.when(s < NS - 1)
        def _():
            pltpu.sync_copy(acc.at[pl.ds(r0, rpt)],
                            agg_hbm.at[c, pl.ds(r0, rpt)])

        @pl.when(s == NS - 1)
        def _():
            pltpu.sync_copy(acc.at[pl.ds(r0, last)],
                            agg_hbm.at[c, pl.ds(r0, last)])

    return spmm_kernel


def _norm_from_deg(deg_ref):
    deg = deg_ref[...]
    return jnp.where(deg > 0.0, lax.rsqrt(jnp.maximum(deg, 1.0)), 0.0)


def _pre_body(x_ref, w_ref, dego_ref, o_ref):
    norm = _norm_from_deg(dego_ref)
    xw = jnp.dot(x_ref[...], w_ref[...], preferred_element_type=jnp.float32,
                 precision=lax.Precision.HIGHEST)
    o_ref[...] = xw * norm


def _mid_body(agg_ref, degi_ref, dego_ref, w_ref, b_ref, o_ref):
    norm_in = _norm_from_deg(degi_ref)
    norm_out = _norm_from_deg(dego_ref)
    h = jax.nn.relu((agg_ref[0] + agg_ref[1]) * norm_in + b_ref[...])
    hw = jnp.dot(h, w_ref[...], preferred_element_type=jnp.float32,
                 precision=lax.Precision.HIGHEST)
    o_ref[...] = hw * norm_out


def _post_body(agg_ref, degi_ref, b_ref, wd1_ref, bd1_ref, wd2_ref, bd2_ref,
               starts_ref, counts_ref, o_ref):
    norm_in = _norm_from_deg(degi_ref)
    h = jax.nn.relu((agg_ref[0] + agg_ref[1]) * norm_in + b_ref[...])
    n = h.shape[0]
    ng = starts_ref.shape[0]
    col = lax.broadcasted_iota(jnp.int32, (ng, n), 1).astype(jnp.float32)
    starts = starts_ref[...]
    counts = counts_ref[...]
    sel = jnp.where((col >= starts) & (col < starts + counts), 1.0, 0.0)
    sums = jnp.dot(sel, h, preferred_element_type=jnp.float32,
                 precision=lax.Precision.HIGHEST)
    means = sums / jnp.maximum(counts, 1.0)
    z = jax.nn.relu(
        jnp.dot(means, wd1_ref[...], preferred_element_type=jnp.float32,
                 precision=lax.Precision.HIGHEST)
        + bd1_ref[...])
    o_ref[...] = (jnp.dot(z, wd2_ref[...], preferred_element_type=jnp.float32,
                 precision=lax.Precision.HIGHEST)
                  + bd2_ref[...])


def kernel(node_features, edge_features, pair_indices, num_nodes, num_edges,
           W_gcn0, b_gcn0, W_gcn1, b_gcn1, W_d1, b_d1, W_d2, b_d2):
    del edge_features, num_edges
    n_nodes, _ = node_features.shape
    n_edges = pair_indices.shape[1]
    hidden = W_gcn0.shape[1]
    ng = num_nodes.shape[0]

    deg_chunk = 128
    spmm_chunk = 80
    assert n_edges % deg_chunk == 0 and n_edges % spmm_chunk == 0

    src = pair_indices[0]
    dst = pair_indices[1]
    zeros_tab = jnp.zeros((n_nodes, hidden), jnp.float32)
    zeros_deg = jnp.zeros((n_nodes,), jnp.float32)

    deg_call = _make_deg_kernel(n_edges, n_nodes, deg_chunk)
    spmm_call = _make_spmm_kernel(n_edges, n_nodes, hidden, spmm_chunk)

    (deg1d,) = deg_call(src, dst, zeros_deg)
    dego_col = deg1d[:n_nodes].reshape(n_nodes, 1)
    degi_col = deg1d[n_nodes:].reshape(n_nodes, 1)

    m0 = pl.pallas_call(
        _pre_body,
        out_shape=jax.ShapeDtypeStruct((n_nodes, hidden), jnp.float32),
    )(node_features, W_gcn0, dego_col)
    (agg0,) = spmm_call(m0, src, dst, zeros_tab)

    m1 = pl.pallas_call(
        _mid_body,
        out_shape=jax.ShapeDtypeStruct((n_nodes, hidden), jnp.float32),
    )(agg0, degi_col, dego_col, W_gcn1, b_gcn0.reshape(1, hidden))
    (agg1,) = spmm_call(m1, src, dst, zeros_tab)

    counts = num_nodes.astype(jnp.float32).reshape(ng, 1)
    starts = jnp.concatenate(
        [jnp.zeros((1,), jnp.float32),
         jnp.cumsum(num_nodes.astype(jnp.float32))[:-1]]).reshape(ng, 1)
    pred_hidden = W_d1.shape[1]
    out = pl.pallas_call(
        _post_body,
        out_shape=jax.ShapeDtypeStruct((ng, 1), jnp.float32),
    )(agg1, degi_col, b_gcn1.reshape(1, hidden), W_d1,
      b_d1.reshape(1, pred_hidden), W_d2, b_d2.reshape(1, 1),
      starts, counts)
    return out
